# Initial kernel scaffold; baseline (speedup 1.0000x reference)
import functools

import numpy as np
import jax
import jax.numpy as jnp
from jax import lax
from jax.experimental import pallas as pl
from jax.experimental.pallas import tpu as pltpu
from jax.experimental.pallas import tpu_sc as plsc

NC, NS, L = 2, 16, 16
NW = NC * NS


@functools.lru_cache(maxsize=None)
def _sc_scores(B, K, D):
    P = K + 1
    b_w = B // NW
    CB = 128
    NCH = b_w // CB
    ROWS = CB * P
    NIDX = ROWS // 128
    NV = D // L

    mesh = plsc.VectorSubcoreMesh(core_axis_name="c", subcore_axis_name="s")

    @functools.partial(
        pl.kernel,
        out_type=jax.ShapeDtypeStruct((B * P, L), jnp.float32),
        mesh=mesh,
        scratch_types=[
            pltpu.VMEM((NCH, 128), jnp.int32),
            pltpu.VMEM((NCH * NIDX, 128), jnp.int32),
            pltpu.VMEM((CB, D), jnp.float32),
            pltpu.VMEM((ROWS, D), jnp.float32),
            pltpu.VMEM((ROWS, L), jnp.float32),
            pltpu.SemaphoreType.DMA,
            pltpu.SemaphoreType.DMA,
        ],
    )
    def k(cen_hbm, ctx_hbm, tw_hbm, cw_hbm, out_hbm,
          cidx, xidx, vbuf, ubuf, part, sem_v, sem_u):
        wid = lax.axis_index("s") * NC + lax.axis_index("c")
        pltpu.sync_copy(cen_hbm.at[pl.ds(wid * NCH, NCH)], cidx)
        pltpu.sync_copy(ctx_hbm.at[pl.ds(wid * (NCH * NIDX), NCH * NIDX)], xidx)
        for c in range(NCH):
            cp_v = pltpu.async_copy(tw_hbm.at[cidx.at[c]], vbuf, sem_v)
            cps = [
                pltpu.async_copy(cw_hbm.at[xidx.at[c * NIDX + t]],
                                 ubuf.at[pl.ds(t * 128, 128)], sem_u)
                for t in range(NIDX)
            ]
            cp_v.wait()
            for cp in cps:
                cp.wait()

            def body(b, carry):
                vr = [vbuf[b, pl.ds(i * L, L)] for i in range(NV)]
                for j in range(P):
                    p = b * P + j
                    acc = vr[0] * ubuf[p, pl.ds(0, L)]
                    for i in range(1, NV):
                        acc = acc + vr[i] * ubuf[p, pl.ds(i * L, L)]
                    part[p, :] = acc
                return carry

            lax.fori_loop(0, CB, body, 0)
            pltpu.sync_copy(part, out_hbm.at[pl.ds((wid * NCH + c) * ROWS, ROWS)])

    return k


@functools.lru_cache(maxsize=None)
def _tc_loss(B, P):
    BLK = 512
    G = B // BLK

    def body(x_ref, m_ref, out_ref):
        x = x_ref[...]
        s = jnp.dot(x, m_ref[...], preferred_element_type=jnp.float32)
        col = lax.broadcasted_iota(jnp.int32, s.shape, 1)
        t = jnp.where(col == 0, s, -s)
        loss = -jnp.log(jax.nn.sigmoid(t) + 1e-09)

        @pl.when(pl.program_id(0) == 0)
        def _():
            out_ref[0, 0] = 0.0

        out_ref[0, 0] += jnp.sum(loss)

        @pl.when(pl.program_id(0) == G - 1)
        def _():
            out_ref[0, 0] = out_ref[0, 0] / B

    return pl.pallas_call(
        body,
        grid=(G,),
        in_specs=[
            pl.BlockSpec((BLK, P * L), lambda i: (i, 0)),
            pl.BlockSpec((P * L, P), lambda i: (0, 0)),
        ],
        out_specs=pl.BlockSpec((1, 1), lambda i: (0, 0)),
        out_shape=jax.ShapeDtypeStruct((1, 1), jnp.float32),
    )


@functools.lru_cache(maxsize=None)
def _lane_sum_matrix(P):
    m = np.zeros((P * L, P), dtype=np.float32)
    for j in range(P):
        m[j * L:(j + 1) * L, j] = 1.0
    return jnp.asarray(m)


def kernel(center_ids, pos_ctx_ids, neg_ctx_ids, target_W, context_W):
    B = center_ids.shape[0]
    K = neg_ctx_ids.shape[1]
    D = target_W.shape[1]
    P = K + 1
    cen = center_ids.astype(jnp.int32).reshape(B // 128, 128)
    ctx = jnp.concatenate(
        [pos_ctx_ids.astype(jnp.int32)[:, None], neg_ctx_ids.astype(jnp.int32)],
        axis=1,
    ).reshape(B * P // 128, 128)
    part = _sc_scores(B, K, D)(cen, ctx, target_W, context_W)
    out = _tc_loss(B, P)(part.reshape(B, P * L), _lane_sum_matrix(P))
    return out[0, 0]


# speedup vs baseline: 2.8087x; 2.8087x over previous
import functools

import numpy as np
import jax
import jax.numpy as jnp
from jax import lax
from jax.experimental import pallas as pl
from jax.experimental.pallas import tpu as pltpu
from jax.experimental.pallas import tpu_sc as plsc

NC, NS, L = 2, 16, 16
NW = NC * NS


def _pad8(n):
    return (n + 7) // 8 * 8


@functools.lru_cache(maxsize=None)
def _sc_scores(B, K, D):
    P = K + 1
    b_w = B // NW
    CB = 128
    NCH = b_w // CB
    ROWS = CB * P
    NIDX = ROWS // 128
    NV = D // L
    CSTRIDE = _pad8(NCH)
    XSTRIDE = _pad8(NCH * NIDX)

    mesh = plsc.VectorSubcoreMesh(
        core_axis_name="c", subcore_axis_name="s", num_cores=NC, num_subcores=NS
    )

    @functools.partial(
        pl.kernel,
        out_type=jax.ShapeDtypeStruct((B, P * L), jnp.float32),
        mesh=mesh,
        compiler_params=pltpu.CompilerParams(use_tc_tiling_on_sc=False),
        scratch_types=[
            pltpu.VMEM((CSTRIDE, 128), jnp.int32),
            pltpu.VMEM((XSTRIDE, 128), jnp.int32),
            pltpu.VMEM((CB, D), jnp.float32),
            pltpu.VMEM((ROWS, D), jnp.float32),
            pltpu.VMEM((CB, P * L), jnp.float32),
            pltpu.SemaphoreType.DMA,
            pltpu.SemaphoreType.DMA,
        ],
    )
    def k(cen_hbm, ctx_hbm, tw_hbm, cw_hbm, out_hbm,
          cidx, xidx, vbuf, ubuf, part, sem_v, sem_u):
        wid = lax.axis_index("s") * NC + lax.axis_index("c")
        pltpu.sync_copy(cen_hbm.at[pl.ds(wid * CSTRIDE, CSTRIDE)], cidx)
        pltpu.sync_copy(ctx_hbm.at[pl.ds(wid * XSTRIDE, XSTRIDE)], xidx)
        for c in range(NCH):
            cp_v = pltpu.async_copy(tw_hbm.at[cidx.at[c]], vbuf, sem_v)
            cps = [
                pltpu.async_copy(cw_hbm.at[xidx.at[c * NIDX + t]],
                                 ubuf.at[pl.ds(t * 128, 128)], sem_u)
                for t in range(NIDX)
            ]
            cp_v.wait()
            for cp in cps:
                cp.wait()

            def body(b, carry):
                vr = [vbuf[b, pl.ds(i * L, L)] for i in range(NV)]
                for j in range(P):
                    p = b * P + j
                    acc = vr[0] * ubuf[p, pl.ds(0, L)]
                    for i in range(1, NV):
                        acc = acc + vr[i] * ubuf[p, pl.ds(i * L, L)]
                    part[b, pl.ds(j * L, L)] = acc
                return carry

            lax.fori_loop(0, CB, body, 0)
            pltpu.sync_copy(part, out_hbm.at[pl.ds((wid * NCH + c) * CB, CB)])

    return k


@functools.lru_cache(maxsize=None)
def _tc_loss(B, P):
    BLK = 512
    G = B // BLK

    def body(x_ref, m_ref, out_ref):
        x = x_ref[...]
        s = jnp.dot(x, m_ref[...], preferred_element_type=jnp.float32)
        col = lax.broadcasted_iota(jnp.int32, s.shape, 1)
        t = jnp.where(col == 0, s, -s)
        loss = -jnp.log(jax.nn.sigmoid(t) + 1e-09)

        @pl.when(pl.program_id(0) == 0)
        def _():
            out_ref[...] = jnp.zeros((1, 1), jnp.float32)

        out_ref[...] = out_ref[...] + jnp.sum(loss)

        @pl.when(pl.program_id(0) == G - 1)
        def _():
            out_ref[...] = out_ref[...] / B

    return pl.pallas_call(
        body,
        grid=(G,),
        in_specs=[
            pl.BlockSpec((BLK, P * L), lambda i: (i, 0)),
            pl.BlockSpec((P * L, P), lambda i: (0, 0)),
        ],
        out_specs=pl.BlockSpec((1, 1), lambda i: (0, 0)),
        out_shape=jax.ShapeDtypeStruct((1, 1), jnp.float32),
    )


@functools.lru_cache(maxsize=None)
def _lane_sum_matrix(P):
    m = np.zeros((P * L, P), dtype=np.float32)
    for j in range(P):
        m[j * L:(j + 1) * L, j] = 1.0
    return jnp.asarray(m)


def _pad_rows(x2d, nw):
    r = x2d.shape[0] // nw
    rp = _pad8(r)
    if rp == r:
        return x2d
    x3 = x2d.reshape(nw, r, 128)
    x3 = jnp.pad(x3, ((0, 0), (0, rp - r), (0, 0)))
    return x3.reshape(nw * rp, 128)


def kernel(center_ids, pos_ctx_ids, neg_ctx_ids, target_W, context_W):
    B = center_ids.shape[0]
    K = neg_ctx_ids.shape[1]
    D = target_W.shape[1]
    P = K + 1
    cen = _pad_rows(center_ids.astype(jnp.int32).reshape(B // 128, 128), NW)
    ctx = jnp.concatenate(
        [pos_ctx_ids.astype(jnp.int32)[:, None], neg_ctx_ids.astype(jnp.int32)],
        axis=1,
    ).reshape(B * P // 128, 128)
    ctx = _pad_rows(ctx, NW)
    part = _sc_scores(B, K, D)(cen, ctx, target_W, context_W)
    out = _tc_loss(B, P)(part, _lane_sum_matrix(P))
    return out[0, 0]
